# baseline (speedup 1.0000x reference)
import functools

import jax
import jax.numpy as jnp
from jax import lax
from jax.experimental import pallas as pl
from jax.experimental.pallas import tpu as pltpu
from jax.experimental.pallas import tpu_sc as plsc

N = 10000
H = 128
E = 320000
NP = 10240
NC = 2
NS = 16
NW = NC * NS
C = 128
FAST_C = 0
GF = 40
GS = 0
CPW = 4 * max(GF, GS)
N_CHUNKS = 16 * 4 * (GF + GS)
E_PAD = N_CHUNKS * C
ROWS_PER_SUB = NP // NS
GATHER_CHUNK = 64
GPW = NP // (NW * GATHER_CHUNK)


def _emb_gather_body(table, idx2d, out, idx_v, rows_v, sem):
    c = lax.axis_index("c")
    s = lax.axis_index("s")
    wid = s * NC + c
    for j in range(GPW):
        row = wid * GPW + j
        pltpu.sync_copy(idx2d.at[row], idx_v)
        pltpu.async_copy(table.at[idx_v], rows_v, sem).wait()
        pltpu.sync_copy(rows_v, out.at[pl.ds(row * GATHER_CHUNK, GATHER_CHUNK)])


_emb_gather = functools.partial(
    pl.kernel,
    mesh=plsc.VectorSubcoreMesh(core_axis_name="c", subcore_axis_name="s"),
    out_type=jax.ShapeDtypeStruct((NP, H), jnp.float32),
    scratch_types=[
        pltpu.VMEM((GATHER_CHUNK,), jnp.int32),
        pltpu.VMEM((GATHER_CHUNK, H), jnp.float32),
        pltpu.SemaphoreType.DMA,
    ],
)(_emb_gather_body)


def _seg_sum_body(m, idxc, out, ij, rows, acc,
                  gsem0, gsem1, ssem0, ssem1, isem0, isem1, isem2, isem3):
    c = lax.axis_index("c")
    s = lax.axis_index("s")
    wid = s * NC + c

    def zrow(row, carry):
        for g in range(H // 16):
            rows[0, row, pl.ds(g * 16, 16)] = jnp.zeros((16,), jnp.float32)
        return carry

    lax.fori_loop(0, C, zrow, 0)
    for t in range(ROWS_PER_SUB // C):
        pltpu.sync_copy(rows.at[0],
                        acc.at[pl.ds(s * ROWS_PER_SUB + t * C, C)])

    gsems = (gsem0, gsem1)
    ssems = (ssem0, ssem1)
    isems = (isem0, isem1, isem2, isem3)

    def i_start(j, i):
        pltpu.async_copy(idxc.at[wid, j], ij.at[i], isems[i])

    def i_wait(j, i):
        pltpu.make_async_copy(idxc.at[wid, j], ij.at[i], isems[i]).wait()

    def g_start(j, r, i):
        pltpu.async_copy(m.at[ij.at[i, 0]], rows.at[r], gsems[r])

    def g_wait(j, r, i):
        pltpu.make_async_copy(m.at[ij.at[i, 0]], rows.at[r], gsems[r]).wait()

    def s_start(j, r, i):
        pltpu.async_copy(rows.at[r], acc.at[ij.at[i, 1]], ssems[r], add=True)

    def s_wait(j, r, i):
        pltpu.make_async_copy(rows.at[r], acc.at[ij.at[i, 1]], ssems[r]).wait()

    plsc.subcore_barrier()

    niter = jnp.where(c == FAST_C, GF, GS)

    @pl.when(niter > 0)
    def _():
        i_start(0, 0)
        i_start(1, 1)
        i_start(2, 2)
        i_wait(0, 0)
        g_start(0, 0, 0)

    def body(q, carry):
        for k in range(4):
            j = q * 4 + k
            r = k & 1
            i = k

            if k == 0:
                @pl.when(q > 0)
                def _():
                    s_wait(j - 1, 1 - r, (i - 1) & 3)
            else:
                s_wait(j - 1, 1 - r, (i - 1) & 3)

            if k == 0:
                i_start(j + 3, (i + 3) & 3)
            else:
                @pl.when(q < niter - 1)
                def _():
                    i_start(j + 3, (i + 3) & 3)

            if k == 3:
                @pl.when(q < niter - 1)
                def _():
                    i_wait(j + 1, (i + 1) & 3)
                    g_start(j + 1, 1 - r, (i + 1) & 3)
            else:
                i_wait(j + 1, (i + 1) & 3)
                g_start(j + 1, 1 - r, (i + 1) & 3)

            g_wait(j, r, i)
            s_start(j, r, i)
        return carry

    lax.fori_loop(0, niter, body, 0)

    @pl.when(niter > 0)
    def _():
        s_wait(0, 1, 3)

    plsc.subcore_barrier()
    pltpu.sync_copy(acc.at[pl.ds(s * ROWS_PER_SUB, ROWS_PER_SUB)],
                    out.at[c, pl.ds(s * ROWS_PER_SUB, ROWS_PER_SUB)])


_seg_sum = functools.partial(
    pl.kernel,
    mesh=plsc.VectorSubcoreMesh(core_axis_name="c", subcore_axis_name="s"),
    out_type=jax.ShapeDtypeStruct((NC, NP, H), jnp.float32),
    scratch_types=[
        pltpu.VMEM((4, 2, C), jnp.int32),
        pltpu.VMEM((2, C, H), jnp.float32),
        pltpu.VMEM_SHARED((NP, H), jnp.float32),
        pltpu.SemaphoreType.DMA,
        pltpu.SemaphoreType.DMA,
        pltpu.SemaphoreType.DMA,
        pltpu.SemaphoreType.DMA,
        pltpu.SemaphoreType.DMA,
        pltpu.SemaphoreType.DMA,
        pltpu.SemaphoreType.DMA,
        pltpu.SemaphoreType.DMA,
    ],
)(_seg_sum_body)


_BLK = 1280
_GRID = NP // _BLK


def _mm_body(x_ref, w_ref, o_ref):
    o_ref[...] = jnp.dot(x_ref[...], w_ref[...], preferred_element_type=jnp.float32)


def _matmul(x, w):
    return pl.pallas_call(
        _mm_body,
        grid=(_GRID,),
        in_specs=[
            pl.BlockSpec((_BLK, H), lambda i: (i, 0)),
            pl.BlockSpec((H, H), lambda i: (0, 0)),
        ],
        out_specs=pl.BlockSpec((_BLK, H), lambda i: (i, 0)),
        out_shape=jax.ShapeDtypeStruct((NP, H), jnp.float32),
    )(x, w)


def _gru_math(p_ref, x_ref, wih_ref, whh_ref, bih_ref, bhh_ref):
    agg = p_ref[0] + p_ref[1]
    x = x_ref[...]
    gi = jnp.dot(agg, wih_ref[...], preferred_element_type=jnp.float32) + bih_ref[...]
    gh = jnp.dot(x, whh_ref[...], preferred_element_type=jnp.float32) + bhh_ref[...]
    r = jax.nn.sigmoid(gi[:, :H] + gh[:, :H])
    z = jax.nn.sigmoid(gi[:, H:2 * H] + gh[:, H:2 * H])
    n = jnp.tanh(gi[:, 2 * H:] + r * gh[:, 2 * H:])
    return (1.0 - z) * n + z * x


def _gru_m_body(p_ref, x_ref, wih_ref, whh_ref, bih_ref, bhh_ref, wn_ref,
                o_ref, m_ref):
    xn = _gru_math(p_ref, x_ref, wih_ref, whh_ref, bih_ref, bhh_ref)
    o_ref[...] = xn
    m_ref[...] = jnp.dot(xn, wn_ref[...], preferred_element_type=jnp.float32)


def _gru_body(p_ref, x_ref, wih_ref, whh_ref, bih_ref, bhh_ref, o_ref):
    o_ref[...] = _gru_math(p_ref, x_ref, wih_ref, whh_ref, bih_ref, bhh_ref)


_P_SPEC = pl.BlockSpec((NC, _BLK, H), lambda i: (0, i, 0))
_X_SPEC = pl.BlockSpec((_BLK, H), lambda i: (i, 0))
_W3_SPEC = pl.BlockSpec((H, 3 * H), lambda i: (0, 0))
_B_SPEC = pl.BlockSpec((1, 3 * H), lambda i: (0, 0))
_W_SPEC = pl.BlockSpec((H, H), lambda i: (0, 0))


def _gru_and_matmul(p, x, wih_t, whh_t, bih, bhh, wn):
    return pl.pallas_call(
        _gru_m_body,
        grid=(_GRID,),
        in_specs=[_P_SPEC, _X_SPEC, _W3_SPEC, _W3_SPEC, _B_SPEC, _B_SPEC, _W_SPEC],
        out_specs=(_X_SPEC, _X_SPEC),
        out_shape=(jax.ShapeDtypeStruct((NP, H), jnp.float32),
                   jax.ShapeDtypeStruct((NP, H), jnp.float32)),
    )(p, x, wih_t, whh_t, bih, bhh, wn)


def _gru(p, x, wih_t, whh_t, bih, bhh):
    return pl.pallas_call(
        _gru_body,
        grid=(_GRID,),
        in_specs=[_P_SPEC, _X_SPEC, _W3_SPEC, _W3_SPEC, _B_SPEC, _B_SPEC],
        out_specs=_X_SPEC,
        out_shape=jax.ShapeDtypeStruct((NP, H), jnp.float32),
    )(p, x, wih_t, whh_t, bih, bhh)


def kernel(inputs, A, emb, ggnn_weight, w_ih, w_hh, b_ih, b_hh):
    inputs_p = jnp.pad(inputs.astype(jnp.int32), (0, NP - N)).reshape(
        NW * GPW, GATHER_CHUNK)
    src_flat = jnp.pad(A[0].astype(jnp.int32), (0, E_PAD - E)).reshape(N_CHUNKS, C)
    dst_flat = jnp.pad(A[1].astype(jnp.int32), (0, E_PAD - E),
                       constant_values=N).reshape(N_CHUNKS, C)
    flat = jnp.stack([src_flat, dst_flat], axis=1)
    fill = jnp.stack([jnp.zeros((C,), jnp.int32),
                      jnp.full((C,), N, jnp.int32)])
    pieces = []
    start = 0
    for wid in range(NW):
        cnt = 4 * (GF if wid % NC == FAST_C else GS)
        block = flat[start:start + cnt]
        start += cnt
        if cnt < CPW:
            block = jnp.concatenate(
                [block, jnp.broadcast_to(fill, (CPW - cnt, 2, C))], axis=0)
        pieces.append(block)
    idx_pack = jnp.stack(pieces)
    wih_t = w_ih.T.reshape(H, 3 * H)
    whh_t = w_hh.T.reshape(H, 3 * H)
    bih = b_ih.reshape(1, 3 * H)
    bhh = b_hh.reshape(1, 3 * H)

    x = _emb_gather(emb, inputs_p)
    m = _matmul(x, ggnn_weight[0])
    p = _seg_sum(m, idx_pack)
    x, m = _gru_and_matmul(p, x, wih_t, whh_t, bih, bhh, ggnn_weight[1])
    p = _seg_sum(m, idx_pack)
    x = _gru(p, x, wih_t, whh_t, bih, bhh)
    return x[:N]


# speedup vs baseline: 1.3769x; 1.3769x over previous
import functools

import jax
import jax.numpy as jnp
from jax import lax
from jax.experimental import pallas as pl
from jax.experimental.pallas import tpu as pltpu
from jax.experimental.pallas import tpu_sc as plsc

N = 10000
H = 128
E = 320000
NP = 10240
NC = 2
NS = 16
NW = NC * NS
C = 128
FAST_C = 0
GF = 29
GS = 11
CPW = 4 * max(GF, GS)
N_CHUNKS = 16 * 4 * (GF + GS)
E_PAD = N_CHUNKS * C
ROWS_PER_SUB = NP // NS
GATHER_CHUNK = 64
GPW = NP // (NW * GATHER_CHUNK)


def _emb_gather_body(table, idx2d, out, idx_v, rows_v, sem):
    c = lax.axis_index("c")
    s = lax.axis_index("s")
    wid = s * NC + c
    for j in range(GPW):
        row = wid * GPW + j
        pltpu.sync_copy(idx2d.at[row], idx_v)
        pltpu.async_copy(table.at[idx_v], rows_v, sem).wait()
        pltpu.sync_copy(rows_v, out.at[pl.ds(row * GATHER_CHUNK, GATHER_CHUNK)])


_emb_gather = functools.partial(
    pl.kernel,
    mesh=plsc.VectorSubcoreMesh(core_axis_name="c", subcore_axis_name="s"),
    out_type=jax.ShapeDtypeStruct((NP, H), jnp.float32),
    scratch_types=[
        pltpu.VMEM((GATHER_CHUNK,), jnp.int32),
        pltpu.VMEM((GATHER_CHUNK, H), jnp.float32),
        pltpu.SemaphoreType.DMA,
    ],
)(_emb_gather_body)


def _seg_sum_body(m, idxc, out, ij, rows, acc,
                  gsem0, gsem1, ssem0, ssem1, isem0, isem1, isem2, isem3):
    c = lax.axis_index("c")
    s = lax.axis_index("s")
    wid = s * NC + c

    def zrow(row, carry):
        for g in range(H // 16):
            rows[0, row, pl.ds(g * 16, 16)] = jnp.zeros((16,), jnp.float32)
        return carry

    lax.fori_loop(0, C, zrow, 0)
    for t in range(ROWS_PER_SUB // C):
        pltpu.sync_copy(rows.at[0],
                        acc.at[pl.ds(s * ROWS_PER_SUB + t * C, C)])

    gsems = (gsem0, gsem1)
    ssems = (ssem0, ssem1)
    isems = (isem0, isem1, isem2, isem3)

    def i_start(j, i):
        pltpu.async_copy(idxc.at[wid, j], ij.at[i], isems[i])

    def i_wait(j, i):
        pltpu.make_async_copy(idxc.at[wid, j], ij.at[i], isems[i]).wait()

    def g_start(j, r, i):
        pltpu.async_copy(m.at[ij.at[i, 0]], rows.at[r], gsems[r])

    def g_wait(j, r, i):
        pltpu.make_async_copy(m.at[ij.at[i, 0]], rows.at[r], gsems[r]).wait()

    def s_start(j, r, i):
        pltpu.async_copy(rows.at[r], acc.at[ij.at[i, 1]], ssems[r], add=True)

    def s_wait(j, r, i):
        pltpu.make_async_copy(rows.at[r], acc.at[ij.at[i, 1]], ssems[r]).wait()

    plsc.subcore_barrier()

    niter = jnp.where(c == FAST_C, GF, GS)

    @pl.when(niter > 0)
    def _():
        i_start(0, 0)
        i_start(1, 1)
        i_start(2, 2)
        i_wait(0, 0)
        g_start(0, 0, 0)

    def body(q, carry):
        for k in range(4):
            j = q * 4 + k
            r = k & 1
            i = k

            if k == 0:
                @pl.when(q > 0)
                def _():
                    s_wait(j - 1, 1 - r, (i - 1) & 3)
            else:
                s_wait(j - 1, 1 - r, (i - 1) & 3)

            if k == 0:
                i_start(j + 3, (i + 3) & 3)
            else:
                @pl.when(q < niter - 1)
                def _():
                    i_start(j + 3, (i + 3) & 3)

            if k == 3:
                @pl.when(q < niter - 1)
                def _():
                    i_wait(j + 1, (i + 1) & 3)
                    g_start(j + 1, 1 - r, (i + 1) & 3)
            else:
                i_wait(j + 1, (i + 1) & 3)
                g_start(j + 1, 1 - r, (i + 1) & 3)

            g_wait(j, r, i)
            s_start(j, r, i)
        return carry

    lax.fori_loop(0, niter, body, 0)

    @pl.when(niter > 0)
    def _():
        s_wait(0, 1, 3)

    plsc.subcore_barrier()
    pltpu.sync_copy(acc.at[pl.ds(s * ROWS_PER_SUB, ROWS_PER_SUB)],
                    out.at[c, pl.ds(s * ROWS_PER_SUB, ROWS_PER_SUB)])


_seg_sum = functools.partial(
    pl.kernel,
    mesh=plsc.VectorSubcoreMesh(core_axis_name="c", subcore_axis_name="s"),
    out_type=jax.ShapeDtypeStruct((NC, NP, H), jnp.float32),
    scratch_types=[
        pltpu.VMEM((4, 2, C), jnp.int32),
        pltpu.VMEM((2, C, H), jnp.float32),
        pltpu.VMEM_SHARED((NP, H), jnp.float32),
        pltpu.SemaphoreType.DMA,
        pltpu.SemaphoreType.DMA,
        pltpu.SemaphoreType.DMA,
        pltpu.SemaphoreType.DMA,
        pltpu.SemaphoreType.DMA,
        pltpu.SemaphoreType.DMA,
        pltpu.SemaphoreType.DMA,
        pltpu.SemaphoreType.DMA,
    ],
)(_seg_sum_body)


_BLK = 1280
_GRID = NP // _BLK


def _mm_body(x_ref, w_ref, o_ref):
    o_ref[...] = jnp.dot(x_ref[...], w_ref[...], preferred_element_type=jnp.float32)


def _matmul(x, w):
    return pl.pallas_call(
        _mm_body,
        grid=(_GRID,),
        in_specs=[
            pl.BlockSpec((_BLK, H), lambda i: (i, 0)),
            pl.BlockSpec((H, H), lambda i: (0, 0)),
        ],
        out_specs=pl.BlockSpec((_BLK, H), lambda i: (i, 0)),
        out_shape=jax.ShapeDtypeStruct((NP, H), jnp.float32),
    )(x, w)


def _gru_math(p_ref, x_ref, wih_ref, whh_ref, bih_ref, bhh_ref):
    agg = p_ref[0] + p_ref[1]
    x = x_ref[...]
    gi = jnp.dot(agg, wih_ref[...], preferred_element_type=jnp.float32) + bih_ref[...]
    gh = jnp.dot(x, whh_ref[...], preferred_element_type=jnp.float32) + bhh_ref[...]
    r = jax.nn.sigmoid(gi[:, :H] + gh[:, :H])
    z = jax.nn.sigmoid(gi[:, H:2 * H] + gh[:, H:2 * H])
    n = jnp.tanh(gi[:, 2 * H:] + r * gh[:, 2 * H:])
    return (1.0 - z) * n + z * x


def _gru_m_body(p_ref, x_ref, wih_ref, whh_ref, bih_ref, bhh_ref, wn_ref,
                o_ref, m_ref):
    xn = _gru_math(p_ref, x_ref, wih_ref, whh_ref, bih_ref, bhh_ref)
    o_ref[...] = xn
    m_ref[...] = jnp.dot(xn, wn_ref[...], preferred_element_type=jnp.float32)


def _gru_body(p_ref, x_ref, wih_ref, whh_ref, bih_ref, bhh_ref, o_ref):
    o_ref[...] = _gru_math(p_ref, x_ref, wih_ref, whh_ref, bih_ref, bhh_ref)


_P_SPEC = pl.BlockSpec((NC, _BLK, H), lambda i: (0, i, 0))
_X_SPEC = pl.BlockSpec((_BLK, H), lambda i: (i, 0))
_W3_SPEC = pl.BlockSpec((H, 3 * H), lambda i: (0, 0))
_B_SPEC = pl.BlockSpec((1, 3 * H), lambda i: (0, 0))
_W_SPEC = pl.BlockSpec((H, H), lambda i: (0, 0))


def _gru_and_matmul(p, x, wih_t, whh_t, bih, bhh, wn):
    return pl.pallas_call(
        _gru_m_body,
        grid=(_GRID,),
        in_specs=[_P_SPEC, _X_SPEC, _W3_SPEC, _W3_SPEC, _B_SPEC, _B_SPEC, _W_SPEC],
        out_specs=(_X_SPEC, _X_SPEC),
        out_shape=(jax.ShapeDtypeStruct((NP, H), jnp.float32),
                   jax.ShapeDtypeStruct((NP, H), jnp.float32)),
    )(p, x, wih_t, whh_t, bih, bhh, wn)


def _gru(p, x, wih_t, whh_t, bih, bhh):
    return pl.pallas_call(
        _gru_body,
        grid=(_GRID,),
        in_specs=[_P_SPEC, _X_SPEC, _W3_SPEC, _W3_SPEC, _B_SPEC, _B_SPEC],
        out_specs=_X_SPEC,
        out_shape=jax.ShapeDtypeStruct((NP, H), jnp.float32),
    )(p, x, wih_t, whh_t, bih, bhh)


def kernel(inputs, A, emb, ggnn_weight, w_ih, w_hh, b_ih, b_hh):
    inputs_p = jnp.pad(inputs.astype(jnp.int32), (0, NP - N)).reshape(
        NW * GPW, GATHER_CHUNK)
    src_flat = jnp.pad(A[0].astype(jnp.int32), (0, E_PAD - E)).reshape(N_CHUNKS, C)
    dst_flat = jnp.pad(A[1].astype(jnp.int32), (0, E_PAD - E),
                       constant_values=N).reshape(N_CHUNKS, C)
    flat = jnp.stack([src_flat, dst_flat], axis=1)
    fill = jnp.stack([jnp.zeros((C,), jnp.int32),
                      jnp.full((C,), N, jnp.int32)])
    pieces = []
    start = 0
    for wid in range(NW):
        cnt = 4 * (GF if wid % NC == FAST_C else GS)
        block = flat[start:start + cnt]
        start += cnt
        if cnt < CPW:
            block = jnp.concatenate(
                [block, jnp.broadcast_to(fill, (CPW - cnt, 2, C))], axis=0)
        pieces.append(block)
    idx_pack = jnp.stack(pieces)
    wih_t = w_ih.T.reshape(H, 3 * H)
    whh_t = w_hh.T.reshape(H, 3 * H)
    bih = b_ih.reshape(1, 3 * H)
    bhh = b_hh.reshape(1, 3 * H)

    x = _emb_gather(emb, inputs_p)
    m = _matmul(x, ggnn_weight[0])
    p = _seg_sum(m, idx_pack)
    x, m = _gru_and_matmul(p, x, wih_t, whh_t, bih, bhh, ggnn_weight[1])
    p = _seg_sum(m, idx_pack)
    x = _gru(p, x, wih_t, whh_t, bih, bhh)
    return x[:N]
